# baseline (speedup 1.0000x reference)
import functools

import jax
import jax.numpy as jnp
from jax import lax
from jax.experimental import pallas as pl
from jax.experimental.pallas import tpu as pltpu
from jax.experimental.pallas import tpu_sc as plsc

_L, _B, _S, _H, _D = 2, 1, 2048, 12, 64
_WINDOW = 128
_K = 1024
_ROW = _H * _D


def _cumsum_lanes(x):
    k = 1
    while k < _S:
        sh = jnp.concatenate([jnp.zeros((1, k), jnp.int32), x[:, :-k]], axis=1)
        x = x + sh
        k *= 2
    return x


def _select_body(attn_ref, kept_ref, csum_ref):
    score = None
    for w in range(3):
        chain = None
        for t in range(16):
            for h in range(4):
                v = attn_ref[0, 4 * w + h, 0, pl.ds(8 * t, 8), :]
                chain = v if chain is None else chain + v
        r4 = chain[0:4] + chain[4:8]
        r2 = r4[0:2] + r4[2:4]
        tw = r2[0:1] + r2[1:2]
        score = tw if score is None else score + tw

    m = jnp.max(score)
    sn = jnp.where(m > 0, score / m, score)

    b = lax.bitcast_convert_type(sn, jnp.int32)
    ks = jnp.where(b < 0, jnp.bitwise_xor(b, jnp.int32(0x7FFFFFFF)), b)

    cntpos = jnp.sum((ks >= 0).astype(jnp.int32))
    t = jnp.where(cntpos >= _K, jnp.int32(0), jnp.int32(-2147483648))
    for bit in range(30, -1, -1):
        cand = jnp.bitwise_or(t, jnp.int32(1 << bit))
        cnt = jnp.sum((ks >= cand).astype(jnp.int32))
        t = jnp.where(cnt >= _K, cand, t)

    gt = ks > t
    cnt_gt = jnp.sum(gt.astype(jnp.int32))
    need = _K - cnt_gt
    eq = ks == t
    rank_eq = _cumsum_lanes(eq.astype(jnp.int32))
    kept = jnp.logical_or(gt, jnp.logical_and(eq, rank_eq <= need))
    kept_i = kept.astype(jnp.int32)
    kept_ref[0] = kept_i
    csum_ref[0] = _cumsum_lanes(kept_i)


def _select(attn, interpret=False):
    kept, csum = pl.pallas_call(
        _select_body,
        grid=(_L,),
        in_specs=[pl.BlockSpec((1, _H, 1, _WINDOW, _S),
                               lambda l: (l, 0, 0, _S // _WINDOW - 1, 0))],
        out_specs=[pl.BlockSpec((1, 1, _S), lambda l: (l, 0, 0)),
                   pl.BlockSpec((1, 1, _S), lambda l: (l, 0, 0))],
        out_shape=[jax.ShapeDtypeStruct((_L, 1, _S), jnp.int32),
                   jax.ShapeDtypeStruct((_L, 1, _S), jnp.int32)],
        interpret=interpret,
    )(attn)
    return kept.reshape(_L, _S), csum.reshape(_L, _S)


@functools.lru_cache(maxsize=1)
def _make_gather():
    mesh = plsc.VectorSubcoreMesh(core_axis_name="c", subcore_axis_name="s")

    @functools.partial(
        pl.kernel,
        mesh=mesh,
        compiler_params=pltpu.CompilerParams(needs_layout_passes=False,
                                             use_tc_tiling_on_sc=True),
        out_type=jax.ShapeDtypeStruct((2 * _L * _K, _ROW), jnp.float32),
        scratch_types=[
            pltpu.VMEM((_S,), jnp.int32),
            pltpu.VMEM((_S,), jnp.int32),
            pltpu.VMEM((128,), jnp.int32),
            pltpu.VMEM((128, _ROW), jnp.float32),
            pltpu.SemaphoreType.DMA,
        ],
    )
    def gather(csum_hbm, kept_hbm, key_tbl, val_tbl, out_hbm,
               c_v, k_v, idx_v, rows_v, sem):
        cid = lax.axis_index("c")
        sid = lax.axis_index("s")
        wid = sid * 2 + cid
        tsel = wid // 16
        r = wid % 16
        l = r // 8
        a = (r % 8) * 128

        pltpu.sync_copy(csum_hbm.at[l], c_v)
        pltpu.sync_copy(kept_hbm.at[l], k_v)

        def body(j, carry):
            c16 = c_v[pl.ds(j * 16, 16)]
            k16 = k_v[pl.ds(j * 16, 16)]
            tok16 = lax.iota(jnp.int32, 16) + (j * 16 + l * _S)
            inb = (k16 == 1) & (c16 > a) & (c16 <= a + 128)
            plsc.store_scatter(idx_v, [c16 - 1 - a], tok16, mask=inb)
            return carry

        lax.fori_loop(0, _S // 16, body, jnp.int32(0))

        @pl.when(tsel == 0)
        def _():
            pltpu.async_copy(key_tbl.at[idx_v], rows_v, sem).wait()

        @pl.when(tsel == 1)
        def _():
            pltpu.async_copy(val_tbl.at[idx_v], rows_v, sem).wait()

        base = tsel * (_L * _K) + l * _K + a
        pltpu.sync_copy(rows_v, out_hbm.at[pl.ds(base, 128)])

    return gather


def kernel(key_cache, value_cache, attention_matrices):
    kept, csum = _select(attention_matrices)
    key_tbl = key_cache.reshape(_L * _S, _ROW)
    val_tbl = value_cache.reshape(_L * _S, _ROW)
    out = _make_gather()(csum, kept, key_tbl, val_tbl)
    out = out.reshape(2, _L, _K, _H, _D)
    return jnp.expand_dims(out, 2)


# speedup vs baseline: 1.2628x; 1.2628x over previous
import functools

import jax
import jax.numpy as jnp
from jax import lax
from jax.experimental import pallas as pl
from jax.experimental.pallas import tpu as pltpu
from jax.experimental.pallas import tpu_sc as plsc

_L, _B, _S, _H, _D = 2, 1, 2048, 12, 64
_WINDOW = 128
_K = 1024
_ROW = _H * _D


def _cumsum_lanes(x):
    k = 1
    while k < _S:
        sh = jnp.concatenate([jnp.zeros((1, k), jnp.int32), x[:, :-k]], axis=1)
        x = x + sh
        k *= 2
    return x


def _select_body(attn_ref, kept_ref, csum_ref):
    score = None
    for w in range(3):
        chain = None
        for t in range(16):
            for h in range(4):
                v = attn_ref[0, 4 * w + h, 0, pl.ds(8 * t, 8), :]
                chain = v if chain is None else chain + v
        r4 = chain[0:4] + chain[4:8]
        r2 = r4[0:2] + r4[2:4]
        tw = r2[0:1] + r2[1:2]
        score = tw if score is None else score + tw

    m = jnp.max(score)
    sn = jnp.where(m > 0, score / m, score)

    b = lax.bitcast_convert_type(sn, jnp.int32)
    ks = jnp.where(b < 0, jnp.bitwise_xor(b, jnp.int32(0x7FFFFFFF)), b)

    cntpos = jnp.sum((ks >= 0).astype(jnp.int32))
    t = jnp.where(cntpos >= _K, jnp.int32(0), jnp.int32(-2147483648))
    for bit in range(30, -1, -1):
        cand = jnp.bitwise_or(t, jnp.int32(1 << bit))
        cnt = jnp.sum((ks >= cand).astype(jnp.int32))
        t = jnp.where(cnt >= _K, cand, t)

    gt = ks > t
    cnt_gt = jnp.sum(gt.astype(jnp.int32))
    need = _K - cnt_gt
    eq = ks == t
    rank_eq = _cumsum_lanes(eq.astype(jnp.int32))
    kept = jnp.logical_or(gt, jnp.logical_and(eq, rank_eq <= need))
    kept_i = kept.astype(jnp.int32)
    kept_ref[0] = kept_i
    csum_ref[0] = _cumsum_lanes(kept_i)


def _select(attn, interpret=False):
    kept, csum = pl.pallas_call(
        _select_body,
        grid=(_L,),
        in_specs=[pl.BlockSpec((1, _H, 1, _WINDOW, _S),
                               lambda l: (l, 0, 0, _S // _WINDOW - 1, 0))],
        out_specs=[pl.BlockSpec((1, 1, _S), lambda l: (l, 0, 0)),
                   pl.BlockSpec((1, 1, _S), lambda l: (l, 0, 0))],
        out_shape=[jax.ShapeDtypeStruct((_L, 1, _S), jnp.int32),
                   jax.ShapeDtypeStruct((_L, 1, _S), jnp.int32)],
        interpret=interpret,
    )(attn)
    return kept.reshape(_L, _S), csum.reshape(_L, _S)


@functools.lru_cache(maxsize=1)
def _make_gather():
    mesh = plsc.VectorSubcoreMesh(core_axis_name="c", subcore_axis_name="s")

    @functools.partial(
        pl.kernel,
        mesh=mesh,
        compiler_params=pltpu.CompilerParams(needs_layout_passes=False),
        out_type=jax.ShapeDtypeStruct((2 * _L * _H * 8, 8 * _K), jnp.float32),
        scratch_types=[
            pltpu.VMEM((_L * _S,), jnp.int32),
            pltpu.VMEM((_L * _S,), jnp.int32),
            pltpu.VMEM((_L * _K,), jnp.int32),
            pltpu.VMEM((16 * 1024,), jnp.float32),
            pltpu.VMEM((8 * _K,), jnp.float32),
            pltpu.SemaphoreType.DMA,
        ],
    )
    def gather(csum_hbm, kept_hbm, key_g, val_g, out_hbm,
               c_v, k_v, tok_v, in_v, out_v, sem):
        cid = lax.axis_index("c")
        sid = lax.axis_index("s")
        wid = sid * 2 + cid

        pltpu.sync_copy(csum_hbm, c_v)
        pltpu.sync_copy(kept_hbm, k_v)

        def compact(j, carry):
            lay = j // 128
            c16 = c_v[pl.ds(j * 16, 16)]
            k16 = k_v[pl.ds(j * 16, 16)]
            tok16 = lax.iota(jnp.int32, 16) + (j % 128) * 16
            plsc.store_scatter(tok_v, [c16 - 1 + lay * _K], tok16,
                               mask=(k16 == 1))
            return carry

        lax.fori_loop(0, _L * _S // 16, compact, jnp.int32(0))

        for tsel, tbl in ((0, key_g), (1, val_g)):
            def row_block(gj, carry, tbl=tbl, tsel=tsel):
                ri = 6 * wid + gj
                tbase = (ri // 96) * _K

                pltpu.sync_copy(tbl.at[ri], in_v)

                def r_body(r, carry2):
                    rrow = r * 2048

                    def chunk_body(c8, carry3):
                        for u in range(8):
                            c = c8 * 8 + u
                            off = tok_v[pl.ds(tbase + c * 16, 16)] + rrow
                            g16 = plsc.load_gather(in_v, [off])
                            out_v[pl.ds(r * 1024 + c * 16, 16)] = g16
                        return carry3

                    lax.fori_loop(0, 8, chunk_body, jnp.int32(0))
                    return carry2

                lax.fori_loop(0, 8, r_body, jnp.int32(0))
                pltpu.sync_copy(out_v, out_hbm.at[tsel * 192 + ri])
                return carry

            lax.fori_loop(0, 6, row_block, jnp.int32(0))

    return gather


def kernel(key_cache, value_cache, attention_matrices):
    kept, csum = _select(attention_matrices)

    def to_rows(cache):
        g = cache.transpose(0, 1, 3, 4, 2)
        g = g.reshape(_L, _B, _H, 8, 8, 16, 128)
        g = g.transpose(0, 1, 2, 4, 3, 5, 6)
        return g.reshape(_L * _H * 8, 8 * _S)

    out = _make_gather()(csum.reshape(_L * _S), kept.reshape(_L * _S),
                         to_rows(key_cache), to_rows(value_cache))
    res = out.reshape(2, _L, _H, 8, 8, _K)
    res = res.transpose(0, 1, 5, 2, 4, 3)
    res = res.reshape(2, _L, _K, _H, _D)
    return jnp.expand_dims(res, 2)
